# baseline (speedup 1.0000x reference)
import jax
import jax.numpy as jnp
from jax import lax
from jax.experimental import pallas as pl
from jax.experimental.pallas import tpu as pltpu
from jax.experimental.pallas import tpu_sc as plsc

N = 4096
D = 64
K = 32
M_DIM = 16
H1 = 258
ROWS = 128
TW = 128
CH = 128
NCHUNK = 8
RC = N // NCHUNK


def _silu(x):
    return x * jax.nn.sigmoid(x)


NO = 256
NCH = N // NO


def _topk_body(coors_ref, coorsT_ref, idx_ref):
    ci = coors_ref[...]
    dx = ci[:, 0:1] - coorsT_ref[0:1, :]
    dy = ci[:, 1:2] - coorsT_ref[1:2, :]
    dz = ci[:, 2:3] - coorsT_ref[2:3, :]
    d = dx * dx + dy * dy + dz * dz
    bits = lax.bitcast_convert_type(d, jnp.int32)
    col = lax.broadcasted_iota(jnp.int32, d.shape, 1)
    keyi = jnp.bitwise_or(
        jnp.bitwise_and(bits + jnp.int32(0x08000000), jnp.int32(-4096)), col)
    key = lax.bitcast_convert_type(keyi, jnp.float32)
    big = jnp.float32(1e30)

    m1 = m2 = m3 = m4 = jnp.full((ROWS, NO), big, jnp.float32)
    for j in range(NCH):
        v = key[:, j * NO:(j + 1) * NO]
        t = jnp.maximum(m1, v)
        m1 = jnp.minimum(m1, v)
        t2 = jnp.maximum(m2, t)
        m2 = jnp.minimum(m2, t)
        t3 = jnp.maximum(m3, t2)
        m3 = jnp.minimum(m3, t2)
        m4 = jnp.minimum(m4, t3)
    lane = lax.broadcasted_iota(jnp.int32, (ROWS, NO), 1)
    picks = []
    m = None
    for _ in range(K):
        m = jnp.min(m1, axis=1, keepdims=True)
        mi32 = lax.bitcast_convert_type(m, jnp.int32)
        picks.append(jnp.bitwise_and(mi32, jnp.int32(4095)))
        o = jnp.bitwise_and(mi32, jnp.int32(NO - 1))
        hit = lane == o
        m1 = jnp.where(hit, m2, m1)
        m2 = jnp.where(hit, m3, m2)
        m3 = jnp.where(hit, m4, m3)
        m4 = jnp.where(hit, big, m4)
    idx_fast = jnp.concatenate(picks, axis=1)

    cnt = jnp.sum((key <= m).astype(jnp.int32), axis=1, keepdims=True)
    bad = jnp.max(jnp.abs(cnt - jnp.int32(K))) > 0

    @pl.when(jnp.logical_not(bad))
    def _():
        idx_ref[...] = idx_fast

    @pl.when(bad)
    def _():
        kk = key
        ps = []
        for _ in range(K):
            mm = jnp.min(kk, axis=1, keepdims=True)
            ps.append(jnp.bitwise_and(
                lax.bitcast_convert_type(mm, jnp.int32), jnp.int32(4095)))
            kk = jnp.where(kk == mm, big, kk)
        idx_ref[...] = jnp.concatenate(ps, axis=1)


def _topk_call(coors_rows, coorsT):
    rows = coors_rows.shape[0]
    return pl.pallas_call(
        _topk_body,
        grid=(rows // ROWS,),
        in_specs=[
            pl.BlockSpec((ROWS, 3), lambda i: (i, 0)),
            pl.BlockSpec((3, N), lambda i: (0, 0)),
        ],
        out_specs=pl.BlockSpec((ROWS, K), lambda i: (i, 0)),
        out_shape=jax.ShapeDtypeStruct((rows, K), jnp.int32),
    )(coors_rows, coorsT)


def _gather_body(tab_hbm, idx_hbm, out_hbm, idx_v, rows_v, sem):
    info = plsc.get_sparse_core_info()
    nc = info.num_cores
    ns = info.num_subcores
    nb = idx_hbm.shape[0]
    bpw = nb // (nc * ns)
    wid = lax.axis_index("s") * nc + lax.axis_index("c")
    base = wid * bpw
    for c in range(bpw // CH):
        off = pl.multiple_of(base + c * CH, CH)
        pltpu.sync_copy(idx_hbm.at[pl.ds(off, CH)], idx_v)
        pltpu.async_copy(tab_hbm.at[idx_v], rows_v, sem).wait()
        pltpu.sync_copy(rows_v, out_hbm.at[pl.ds(off, CH)])


def _gather_call(table, idx_flat):
    mesh = plsc.VectorSubcoreMesh(core_axis_name="c", subcore_axis_name="s")
    fn = pl.kernel(
        _gather_body,
        mesh=mesh,
        out_type=jax.ShapeDtypeStruct((idx_flat.shape[0], TW), jnp.float32),
        scratch_types=[
            pltpu.VMEM((CH,), jnp.int32),
            pltpu.VMEM((CH, TW), jnp.float32),
            pltpu.SemaphoreType.DMA,
        ],
    )
    return fn(table, idx_flat)


def _mlp_body(G, feats, coors, We1a, We1bd, be1, We2, be2,
              Wc1, bc1, Wc2, bc2, Wn1, bn1, Wn2, bn2, lng, lnb,
              node_ref, coors_ref):
    g = G[...]
    fj = g[:, 0:D]
    cj = g[:, D:D + 3].reshape(ROWS, K, 3)
    f = feats[...]
    c = coors[...]
    rel = c[:, None, :] - cj
    dist = jnp.sum(rel * rel, axis=-1)

    h = jnp.dot(fj, We1bd[0:D, :], preferred_element_type=jnp.float32)
    pi = jnp.dot(f, We1a[...], preferred_element_type=jnp.float32) + be1[...]
    h3 = (h.reshape(ROWS, K, H1) + pi[:, None, :]
          + dist[:, :, None] * We1bd[D:D + 1, :].reshape(1, 1, H1))
    h3 = _silu(h3)
    m = _silu(jnp.dot(h3.reshape(ROWS * K, H1), We2[...],
                      preferred_element_type=jnp.float32) + be2[...])
    cw = jnp.dot(_silu(jnp.dot(m, Wc1[...],
                               preferred_element_type=jnp.float32) + bc1[...]),
                 Wc2[...], preferred_element_type=jnp.float32) + bc2[...]
    w3 = cw.reshape(ROWS, K, 1)
    coors_ref[...] = jnp.sum(w3 * rel, axis=1) + c

    mi = jnp.sum(m.reshape(ROWS, K, M_DIM), axis=1)
    mu = jnp.mean(f, axis=-1, keepdims=True)
    var = jnp.mean((f - mu) ** 2, axis=-1, keepdims=True)
    normed = (f - mu) / jnp.sqrt(var + 1e-5) * lng[...] + lnb[...]
    ni = jnp.concatenate([normed, mi], axis=-1)
    hn = _silu(jnp.dot(ni, Wn1[...], preferred_element_type=jnp.float32)
               + bn1[...])
    node_ref[...] = (jnp.dot(hn, Wn2[...], preferred_element_type=jnp.float32)
                     + bn2[...] + f)


def _mlp_call(G, feats2, coors2, We1a, We1bd, be1, We2, be2,
              Wc1, bc1, Wc2, bc2, Wn1, bn1, Wn2, bn2, lng, lnb):
    rows = feats2.shape[0]
    full = lambda shape: pl.BlockSpec(shape, lambda i: tuple(0 for _ in shape))
    return pl.pallas_call(
        _mlp_body,
        grid=(rows // ROWS,),
        in_specs=[
            pl.BlockSpec((ROWS * K, TW), lambda i: (i, 0)),
            pl.BlockSpec((ROWS, D), lambda i: (i, 0)),
            pl.BlockSpec((ROWS, 3), lambda i: (i, 0)),
            full((D, H1)), full((D + 1, H1)), full((1, H1)),
            full((H1, M_DIM)), full((1, M_DIM)),
            full((M_DIM, M_DIM * 4)), full((1, M_DIM * 4)),
            full((M_DIM * 4, 1)), full((1, 1)),
            full((D + M_DIM, D * 2)), full((1, D * 2)),
            full((D * 2, D)), full((1, D)),
            full((1, D)), full((1, D)),
        ],
        out_specs=[
            pl.BlockSpec((ROWS, D), lambda i: (i, 0)),
            pl.BlockSpec((ROWS, 3), lambda i: (i, 0)),
        ],
        out_shape=[
            jax.ShapeDtypeStruct((rows, D), jnp.float32),
            jax.ShapeDtypeStruct((rows, 3), jnp.float32),
        ],
    )(G, feats2, coors2, We1a, We1bd, be1, We2, be2,
      Wc1, bc1, Wc2, bc2, Wn1, bn1, Wn2, bn2, lng, lnb)


def kernel(feats, coors, mask, We1, be1, We2, be2, Wc1, bc1, Wc2, bc2,
           Wn1, bn1, Wn2, bn2, ln_g, ln_b):
    del mask
    f2 = feats[0]
    c2 = coors[0]
    coorsT = c2.T
    table = jnp.concatenate(
        [f2, c2, jnp.zeros((N, TW - D - 3), jnp.float32)], axis=1)

    We1a = We1[:D, :]
    We1bd = We1[D:2 * D + 1, :]
    wargs = (We1a, We1bd, be1.reshape(1, H1), We2,
             be2.reshape(1, M_DIM), Wc1, bc1.reshape(1, M_DIM * 4), Wc2,
             bc2.reshape(1, 1), Wn1, bn1.reshape(1, D * 2), Wn2,
             bn2.reshape(1, D), ln_g.reshape(1, D), ln_b.reshape(1, D))

    idxs = [_topk_call(lax.slice(c2, (c * RC, 0), ((c + 1) * RC, 3)), coorsT)
            for c in range(NCHUNK)]
    Gs = [_gather_call(table, idxs[c].reshape(RC * K)) for c in range(NCHUNK)]
    outs = [_mlp_call(Gs[c],
                      lax.slice(f2, (c * RC, 0), ((c + 1) * RC, D)),
                      lax.slice(c2, (c * RC, 0), ((c + 1) * RC, 3)),
                      *wargs)
            for c in range(NCHUNK)]
    node2 = jnp.concatenate([o[0] for o in outs], axis=0)
    coors_out = jnp.concatenate([o[1] for o in outs], axis=0)
    return node2.reshape(1, N, D), coors_out.reshape(1, N, 3)


# speedup vs baseline: 1.1234x; 1.1234x over previous
import jax
import jax.numpy as jnp
from jax import lax
from jax.experimental import pallas as pl
from jax.experimental.pallas import tpu as pltpu
from jax.experimental.pallas import tpu_sc as plsc

N = 4096
D = 64
K = 32
M_DIM = 16
H1 = 258
ROWS = 128
TW = 128
CH = 128
NCHUNK = 2
RC = N // NCHUNK


def _silu(x):
    return x * jax.nn.sigmoid(x)


NO = 256
NCH = N // NO


def _topk_body(coors_ref, coorsT_ref, idx_ref):
    ci = coors_ref[...]
    dx = ci[:, 0:1] - coorsT_ref[0:1, :]
    dy = ci[:, 1:2] - coorsT_ref[1:2, :]
    dz = ci[:, 2:3] - coorsT_ref[2:3, :]
    d = dx * dx + dy * dy + dz * dz
    bits = lax.bitcast_convert_type(d, jnp.int32)
    col = lax.broadcasted_iota(jnp.int32, d.shape, 1)
    keyi = jnp.bitwise_or(
        jnp.bitwise_and(bits + jnp.int32(0x08000000), jnp.int32(-4096)), col)
    key = lax.bitcast_convert_type(keyi, jnp.float32)
    big = jnp.float32(1e30)

    m1 = m2 = m3 = m4 = jnp.full((ROWS, NO), big, jnp.float32)
    for j in range(NCH):
        v = key[:, j * NO:(j + 1) * NO]
        t = jnp.maximum(m1, v)
        m1 = jnp.minimum(m1, v)
        t2 = jnp.maximum(m2, t)
        m2 = jnp.minimum(m2, t)
        t3 = jnp.maximum(m3, t2)
        m3 = jnp.minimum(m3, t2)
        m4 = jnp.minimum(m4, t3)
    lane = lax.broadcasted_iota(jnp.int32, (ROWS, NO), 1)
    picks = []
    m = None
    for _ in range(K):
        m = jnp.min(m1, axis=1, keepdims=True)
        mi32 = lax.bitcast_convert_type(m, jnp.int32)
        picks.append(jnp.bitwise_and(mi32, jnp.int32(4095)))
        o = jnp.bitwise_and(mi32, jnp.int32(NO - 1))
        hit = lane == o
        m1 = jnp.where(hit, m2, m1)
        m2 = jnp.where(hit, m3, m2)
        m3 = jnp.where(hit, m4, m3)
        m4 = jnp.where(hit, big, m4)
    idx_fast = jnp.concatenate(picks, axis=1)

    cnt = jnp.sum((key <= m).astype(jnp.int32), axis=1, keepdims=True)
    bad = jnp.max(jnp.abs(cnt - jnp.int32(K))) > 0

    @pl.when(jnp.logical_not(bad))
    def _():
        idx_ref[...] = idx_fast

    @pl.when(bad)
    def _():
        kk = key
        ps = []
        for _ in range(K):
            mm = jnp.min(kk, axis=1, keepdims=True)
            ps.append(jnp.bitwise_and(
                lax.bitcast_convert_type(mm, jnp.int32), jnp.int32(4095)))
            kk = jnp.where(kk == mm, big, kk)
        idx_ref[...] = jnp.concatenate(ps, axis=1)


def _topk_call(coors_rows, coorsT):
    rows = coors_rows.shape[0]
    return pl.pallas_call(
        _topk_body,
        grid=(rows // ROWS,),
        in_specs=[
            pl.BlockSpec((ROWS, 3), lambda i: (i, 0)),
            pl.BlockSpec((3, N), lambda i: (0, 0)),
        ],
        out_specs=pl.BlockSpec((ROWS, K), lambda i: (i, 0)),
        out_shape=jax.ShapeDtypeStruct((rows, K), jnp.int32),
    )(coors_rows, coorsT)


def _gather_body(tab_hbm, idx_hbm, out_hbm, idx_v, rows_v, sem):
    info = plsc.get_sparse_core_info()
    nc = info.num_cores
    ns = info.num_subcores
    nb = idx_hbm.shape[0]
    bpw = nb // (nc * ns)
    wid = lax.axis_index("s") * nc + lax.axis_index("c")
    base = wid * bpw
    for c in range(bpw // CH):
        off = pl.multiple_of(base + c * CH, CH)
        pltpu.sync_copy(idx_hbm.at[pl.ds(off, CH)], idx_v)
        pltpu.async_copy(tab_hbm.at[idx_v], rows_v, sem).wait()
        pltpu.sync_copy(rows_v, out_hbm.at[pl.ds(off, CH)])


def _gather_call(table, idx_flat):
    mesh = plsc.VectorSubcoreMesh(core_axis_name="c", subcore_axis_name="s")
    fn = pl.kernel(
        _gather_body,
        mesh=mesh,
        out_type=jax.ShapeDtypeStruct((idx_flat.shape[0], TW), jnp.float32),
        scratch_types=[
            pltpu.VMEM((CH,), jnp.int32),
            pltpu.VMEM((CH, TW), jnp.float32),
            pltpu.SemaphoreType.DMA,
        ],
    )
    return fn(table, idx_flat)


def _mlp_body(G, feats, coors, We1a, We1bd, be1, We2, be2,
              Wc1, bc1, Wc2, bc2, Wn1, bn1, Wn2, bn2, lng, lnb,
              node_ref, coors_ref):
    g = G[...]
    fj = g[:, 0:D]
    cj = g[:, D:D + 3].reshape(ROWS, K, 3)
    f = feats[...]
    c = coors[...]
    rel = c[:, None, :] - cj
    dist = jnp.sum(rel * rel, axis=-1)

    h = jnp.dot(fj, We1bd[0:D, :], preferred_element_type=jnp.float32)
    pi = jnp.dot(f, We1a[...], preferred_element_type=jnp.float32) + be1[...]
    h3 = (h.reshape(ROWS, K, H1) + pi[:, None, :]
          + dist[:, :, None] * We1bd[D:D + 1, :].reshape(1, 1, H1))
    h3 = _silu(h3)
    m = _silu(jnp.dot(h3.reshape(ROWS * K, H1), We2[...],
                      preferred_element_type=jnp.float32) + be2[...])
    cw = jnp.dot(_silu(jnp.dot(m, Wc1[...],
                               preferred_element_type=jnp.float32) + bc1[...]),
                 Wc2[...], preferred_element_type=jnp.float32) + bc2[...]
    w3 = cw.reshape(ROWS, K, 1)
    coors_ref[...] = jnp.sum(w3 * rel, axis=1) + c

    mi = jnp.sum(m.reshape(ROWS, K, M_DIM), axis=1)
    mu = jnp.mean(f, axis=-1, keepdims=True)
    var = jnp.mean((f - mu) ** 2, axis=-1, keepdims=True)
    normed = (f - mu) / jnp.sqrt(var + 1e-5) * lng[...] + lnb[...]
    ni = jnp.concatenate([normed, mi], axis=-1)
    hn = _silu(jnp.dot(ni, Wn1[...], preferred_element_type=jnp.float32)
               + bn1[...])
    node_ref[...] = (jnp.dot(hn, Wn2[...], preferred_element_type=jnp.float32)
                     + bn2[...] + f)


def _mlp_call(G, feats2, coors2, We1a, We1bd, be1, We2, be2,
              Wc1, bc1, Wc2, bc2, Wn1, bn1, Wn2, bn2, lng, lnb):
    rows = feats2.shape[0]
    full = lambda shape: pl.BlockSpec(shape, lambda i: tuple(0 for _ in shape))
    return pl.pallas_call(
        _mlp_body,
        grid=(rows // ROWS,),
        in_specs=[
            pl.BlockSpec((ROWS * K, TW), lambda i: (i, 0)),
            pl.BlockSpec((ROWS, D), lambda i: (i, 0)),
            pl.BlockSpec((ROWS, 3), lambda i: (i, 0)),
            full((D, H1)), full((D + 1, H1)), full((1, H1)),
            full((H1, M_DIM)), full((1, M_DIM)),
            full((M_DIM, M_DIM * 4)), full((1, M_DIM * 4)),
            full((M_DIM * 4, 1)), full((1, 1)),
            full((D + M_DIM, D * 2)), full((1, D * 2)),
            full((D * 2, D)), full((1, D)),
            full((1, D)), full((1, D)),
        ],
        out_specs=[
            pl.BlockSpec((ROWS, D), lambda i: (i, 0)),
            pl.BlockSpec((ROWS, 3), lambda i: (i, 0)),
        ],
        out_shape=[
            jax.ShapeDtypeStruct((rows, D), jnp.float32),
            jax.ShapeDtypeStruct((rows, 3), jnp.float32),
        ],
    )(G, feats2, coors2, We1a, We1bd, be1, We2, be2,
      Wc1, bc1, Wc2, bc2, Wn1, bn1, Wn2, bn2, lng, lnb)


def kernel(feats, coors, mask, We1, be1, We2, be2, Wc1, bc1, Wc2, bc2,
           Wn1, bn1, Wn2, bn2, ln_g, ln_b):
    del mask
    f2 = feats[0]
    c2 = coors[0]
    coorsT = c2.T
    table = jnp.concatenate(
        [f2, c2, jnp.zeros((N, TW - D - 3), jnp.float32)], axis=1)

    We1a = We1[:D, :]
    We1bd = We1[D:2 * D + 1, :]
    wargs = (We1a, We1bd, be1.reshape(1, H1), We2,
             be2.reshape(1, M_DIM), Wc1, bc1.reshape(1, M_DIM * 4), Wc2,
             bc2.reshape(1, 1), Wn1, bn1.reshape(1, D * 2), Wn2,
             bn2.reshape(1, D), ln_g.reshape(1, D), ln_b.reshape(1, D))

    idxs = [_topk_call(lax.slice(c2, (c * RC, 0), ((c + 1) * RC, 3)), coorsT)
            for c in range(NCHUNK)]
    Gs = [_gather_call(table, idxs[c].reshape(RC * K)) for c in range(NCHUNK)]
    outs = [_mlp_call(Gs[c],
                      lax.slice(f2, (c * RC, 0), ((c + 1) * RC, D)),
                      lax.slice(c2, (c * RC, 0), ((c + 1) * RC, 3)),
                      *wargs)
            for c in range(NCHUNK)]
    node2 = jnp.concatenate([o[0] for o in outs], axis=0)
    coors_out = jnp.concatenate([o[1] for o in outs], axis=0)
    return node2.reshape(1, N, D), coors_out.reshape(1, N, 3)
